# Initial kernel scaffold; baseline (speedup 1.0000x reference)
import jax
import jax.numpy as jnp
from jax.experimental import pallas as pl


def kernel(user, u_ir, nbr, item, rate, user_emb, item_emb, edge_index, edge_weight):
    raise NotImplementedError("write your pallas kernel here")


# speedup vs baseline: 5.8534x; 5.8534x over previous
import dataclasses
import functools

import jax
import jax.numpy as jnp
from jax import lax
from jax.experimental import pallas as pl
from jax.experimental.pallas import tpu as pltpu
from jax.experimental.pallas import tpu_sc as plsc

N_U = 15000
N_I = 35000
NN = N_U + N_I
NN_PAD = 50048
D = 64
DH = 32
LAYERS = 3
BATCH = 4096

TILES = 16
CHUNK = 128
CHUNKS_PER_TILE = 392
E_PAD = CHUNKS_PER_TILE * CHUNK * TILES

ROWS_PER_TILE = NN_PAD // TILES
ZROWS = 391
PB = BATCH // 32

_f32 = jnp.float32
_i32 = jnp.int32


def _mesh():
  return plsc.VectorSubcoreMesh(core_axis_name="c", subcore_axis_name="s")


def _sc_params():
  cp = pltpu.CompilerParams()
  fields = pltpu.CompilerParams.__dataclass_fields__
  if "needs_layout_passes" in fields:
    cp = dataclasses.replace(cp, needs_layout_passes=False)
  if "use_tc_tiling_on_sc" in fields:
    cp = dataclasses.replace(cp, use_tc_tiling_on_sc=False)
  return cp


def _propagate(t0, srcs, dsts, wts):
  out_t = tuple(jax.ShapeDtypeStruct((2 * NN_PAD, DH), _f32) for _ in range(LAYERS))

  @functools.partial(
      pl.kernel,
      out_type=out_t,
      mesh=_mesh(),
      compiler_params=_sc_params(),
      scratch_types=[
          pltpu.VMEM_SHARED((NN_PAD, DH), _f32),
          pltpu.VMEM((ZROWS, DH), _f32),
          pltpu.VMEM((2, CHUNK), _i32),
          pltpu.VMEM((2, CHUNK), _i32),
          pltpu.VMEM((2, CHUNK), _i32),
          pltpu.VMEM((2, CHUNK), _i32),
          pltpu.VMEM((2, CHUNK), _f32),
          pltpu.VMEM((2, CHUNK), _f32),
          pltpu.VMEM((CHUNK, DH), _f32),
          pltpu.VMEM((CHUNK, DH), _f32),
          pltpu.SemaphoreType.DMA,
          pltpu.SemaphoreType.DMA,
          pltpu.SemaphoreType.DMA,
          pltpu.SemaphoreType.DMA,
          pltpu.SemaphoreType.DMA,
          pltpu.SemaphoreType.DMA,
      ],
  )
  def k(t0r, srcr, dstr, wr, t1r, t2r, t3r, acc, zbuf,
        s0, s1, d0, d1, w0, w1, m0, m1, is0, is1, gs0, gs1, ss0, ss1):
    sb = (s0, s1)
    db = (d0, d1)
    wb = (w0, w1)
    mb = (m0, m1)
    isem = (is0, is1)
    gsem = (gs0, gs1)
    ssem = (ss0, ss1)

    cid = lax.axis_index("c")
    sid = lax.axis_index("s")
    row0 = sid * ROWS_PER_TILE
    src_off = cid * NN_PAD

    @pl.loop(0, ZROWS)
    def _(r):
      for h in range(2):
        zbuf[r, pl.ds(h * 16, 16)] = jnp.zeros((16,), _f32)

    for z in range(ROWS_PER_TILE // ZROWS):
      pltpu.sync_copy(zbuf, acc.at[pl.ds(row0 + z * ZROWS, ZROWS)])
    plsc.subcore_barrier()

    def idx_issue(i, s):
      row = sid * CHUNKS_PER_TILE + 2 * i
      pltpu.async_copy(srcr.at[pl.ds(row, 2)], sb[s], isem[s])
      pltpu.async_copy(dstr.at[pl.ds(row, 2)], db[s], isem[s])
      pltpu.async_copy(wr.at[pl.ds(row, 2)], wb[s], isem[s])

    def idx_wait(s):
      pltpu.make_async_copy(srcr.at[pl.ds(0, 2)], sb[s], isem[s]).wait()
      pltpu.make_async_copy(dstr.at[pl.ds(0, 2)], db[s], isem[s]).wait()
      pltpu.make_async_copy(wr.at[pl.ds(0, 2)], wb[s], isem[s]).wait()

    def scale(s, half, mref):
      @pl.loop(0, CHUNK, step=4)
      def _(j0):
        for dj in range(4):
          j = j0 + dj
          wsplat = plsc.load_gather(
              wb[s], [jnp.full((16,), half, _i32),
                      jnp.full((16,), j, _i32)])
          for h in range(2):
            sl = pl.ds(h * 16, 16)
            mref[j, sl] = mref[j, sl] * wsplat

    def pair_step(i, s, tprev, prefetch=True):
      ns = 1 - s
      idx_wait(s)
      for r in range(2):
        for kk in range(CHUNK // 16):
          sl = pl.ds(kk * 16, 16)
          sb[s][r, sl] = sb[s][r, sl] + src_off
      ga = pltpu.async_copy(tprev.at[sb[s].at[0]], mb[0], gsem[0])
      gb = pltpu.async_copy(tprev.at[sb[s].at[1]], mb[1], gsem[1])
      if prefetch:
        idx_issue(i + 1, ns)
      ga.wait()
      scale(s, 0, mb[0])
      sa = pltpu.async_copy(mb[0], acc.at[db[s].at[0]], ssem[0], add=True)
      gb.wait()
      scale(s, 1, mb[1])
      sc = pltpu.async_copy(mb[1], acc.at[db[s].at[1]], ssem[1], add=True)
      sa.wait()
      sc.wait()

    NPAIR = CHUNKS_PER_TILE // 2

    def layer(tprev, tout):
      idx_issue(0, 0)

      @pl.loop(0, (NPAIR - 2) // 2)
      def _(kk):
        pair_step(2 * kk, 0, tprev)
        pair_step(2 * kk + 1, 1, tprev)

      pair_step(NPAIR - 2, 0, tprev)
      pair_step(NPAIR - 1, 1, tprev, prefetch=False)
      plsc.subcore_barrier()
      pltpu.sync_copy(acc.at[pl.ds(row0, ROWS_PER_TILE)],
                      tout.at[pl.ds(src_off + row0, ROWS_PER_TILE)])
      for z in range(ROWS_PER_TILE // ZROWS):
        pltpu.sync_copy(zbuf, acc.at[pl.ds(row0 + z * ZROWS, ZROWS)])
      plsc.subcore_barrier()

    layer(t0r, t1r)
    layer(t1r, t2r)
    layer(t2r, t3r)

  return k(t0, srcs, dsts, wts)


def _readout(user, item, user_emb, item_emb, t0, t1, t2, t3):
  out_t = (
      jax.ShapeDtypeStruct((BATCH, DH), _f32),
      jax.ShapeDtypeStruct((BATCH, DH), _f32),
      jax.ShapeDtypeStruct((BATCH, DH), _f32),
      jax.ShapeDtypeStruct((BATCH, DH), _f32),
      jax.ShapeDtypeStruct((BATCH, D), _f32),
      jax.ShapeDtypeStruct((BATCH, D), _f32),
  )

  @functools.partial(
      pl.kernel,
      out_type=out_t,
      mesh=_mesh(),
      compiler_params=_sc_params(),
      scratch_types=[
          pltpu.VMEM((PB,), _i32),
          pltpu.VMEM((PB,), _i32),
          pltpu.VMEM((PB,), _i32),
          pltpu.VMEM((PB,), _i32),
          pltpu.VMEM((PB, DH), _f32),
          pltpu.VMEM((PB, DH), _f32),
          pltpu.VMEM((PB, DH), _f32),
          pltpu.VMEM((PB, DH), _f32),
          pltpu.VMEM((PB, D), _f32),
          pltpu.VMEM((PB, D), _f32),
          pltpu.VMEM((PB, DH), _f32),
      ],
  )
  def k(userr, itemr, uer, ier, t0r, t1r, t2r, t3r,
        uslr, ushr, islr, ishr, uegor, iegor,
        ub, ubh, ib, ibh, su_lo, su_hi, si_lo, si_hi, eu, ei, tmp):
    cid = lax.axis_index("c")
    sid = lax.axis_index("s")
    wid = cid * TILES + sid
    base = wid * PB

    pltpu.sync_copy(userr.at[pl.ds(base, PB)], ub)
    pltpu.sync_copy(itemr.at[pl.ds(base, PB)], ib)

    pltpu.sync_copy(uer.at[ub], eu)
    pltpu.sync_copy(ier.at[ib], ei)

    for kk in range(PB // 16):
      sl = pl.ds(kk * 16, 16)
      ubh[sl] = ub[sl] + NN_PAD
      ib[sl] = ib[sl] + N_U
      ibh[sl] = ib[sl] + NN_PAD

    for idx, acc in ((ub, su_lo), (ubh, su_hi), (ib, si_lo), (ibh, si_hi)):
      pltpu.sync_copy(t0r.at[idx], acc)
      for tr in (t1r, t2r, t3r):
        pltpu.sync_copy(tr.at[idx], tmp)

        @pl.loop(0, PB)
        def _(r):
          for h in range(2):
            sl = pl.ds(h * 16, 16)
            acc[r, sl] = acc[r, sl] + tmp[r, sl]

    pltpu.sync_copy(su_lo, uslr.at[pl.ds(base, PB)])
    pltpu.sync_copy(su_hi, ushr.at[pl.ds(base, PB)])
    pltpu.sync_copy(si_lo, islr.at[pl.ds(base, PB)])
    pltpu.sync_copy(si_hi, ishr.at[pl.ds(base, PB)])
    pltpu.sync_copy(eu, uegor.at[pl.ds(base, PB)])
    pltpu.sync_copy(ei, iegor.at[pl.ds(base, PB)])

  return k(user, item, user_emb, item_emb, t0, t1, t2, t3)


def _dot_tc(usl, ush, isl, ish):
  def body(a, b, c, d, o):
    o[...] = jnp.sum(a[...] * c[...] + b[...] * d[...], axis=1,
                     keepdims=True) * jnp.float32(1.0 / 16.0)

  out = pl.pallas_call(
      body,
      out_shape=jax.ShapeDtypeStruct((BATCH, 1), _f32),
  )(usl, ush, isl, ish)
  return out.reshape(BATCH)


def kernel(user, u_ir, nbr, item, rate, user_emb, item_emb, edge_index,
           edge_weight):
  del u_ir, nbr, rate
  zrow = jnp.zeros((NN_PAD - NN, DH), _f32)
  t0 = jnp.concatenate(
      [user_emb[:, :DH], item_emb[:, :DH], zrow,
       user_emb[:, DH:], item_emb[:, DH:], zrow],
      axis=0)
  dst = edge_index[0]
  src = edge_index[1]
  e = src.shape[0]
  pad = E_PAD - e
  srcp = jnp.concatenate([src, jnp.zeros((pad,), _i32)]).reshape(-1, CHUNK)
  dstp = jnp.concatenate([dst, jnp.zeros((pad,), _i32)]).reshape(-1, CHUNK)
  wp = jnp.concatenate([edge_weight, jnp.zeros((pad,), _f32)]).reshape(-1, CHUNK)

  t1, t2, t3 = _propagate(t0, srcp, dstp, wp)
  usl, ush, isl, ish, uego, iego = _readout(
      user, item, user_emb, item_emb, t0, t1, t2, t3)
  pred = _dot_tc(usl, ush, isl, ish)
  return pred, uego, iego


# speedup vs baseline: 8.1187x; 1.3870x over previous
import dataclasses
import functools

import jax
import jax.numpy as jnp
from jax import lax
from jax.experimental import pallas as pl
from jax.experimental.pallas import tpu as pltpu
from jax.experimental.pallas import tpu_sc as plsc

N_U = 15000
N_I = 35000
NN = N_U + N_I
NN_PAD = 50048
D = 64
DH = 32
LAYERS = 3
BATCH = 4096

TILES = 16
CHUNK = 128
CHUNKS_PER_TILE = 392
E_PAD = CHUNKS_PER_TILE * CHUNK * TILES

ROWS_PER_TILE = NN_PAD // TILES
SROWS = 184
PB = BATCH // 32

_f32 = jnp.float32
_i32 = jnp.int32


def _mesh():
  return plsc.VectorSubcoreMesh(core_axis_name="c", subcore_axis_name="s")


def _sc_params():
  cp = pltpu.CompilerParams()
  fields = pltpu.CompilerParams.__dataclass_fields__
  if "needs_layout_passes" in fields:
    cp = dataclasses.replace(cp, needs_layout_passes=False)
  if "use_tc_tiling_on_sc" in fields:
    cp = dataclasses.replace(cp, use_tc_tiling_on_sc=False)
  return cp


def _rsqrt16(x):
  i = plsc.bitcast(x, _i32)
  i = jnp.full((16,), 0x5F3759DF, _i32) - lax.shift_right_logical(i, 1)
  y = plsc.bitcast(i, _f32)
  half = x * jnp.float32(0.5)
  for _ in range(3):
    y = y * (jnp.float32(1.5) - half * y * y)
  return y


def _propagate(t0, srcs, dsts, zcol, zslab):
  out_t = (
      tuple(jax.ShapeDtypeStruct((2 * NN_PAD, DH), _f32) for _ in range(4))
      + (jax.ShapeDtypeStruct((NN_PAD,), _f32),))

  @functools.partial(
      pl.kernel,
      out_type=out_t,
      mesh=_mesh(),
      compiler_params=_sc_params(),
      scratch_types=[
          pltpu.VMEM_SHARED((NN_PAD, DH), _f32),
          pltpu.VMEM_SHARED((NN_PAD,), _f32),
          pltpu.VMEM((2, CHUNK), _i32),
          pltpu.VMEM((2, CHUNK), _i32),
          pltpu.VMEM((2, CHUNK), _i32),
          pltpu.VMEM((2, CHUNK), _i32),
          pltpu.VMEM((CHUNK, DH), _f32),
          pltpu.VMEM((CHUNK, DH), _f32),
          pltpu.VMEM((CHUNK,), _f32),
          pltpu.VMEM((SROWS, DH), _f32),
          pltpu.VMEM((192,), _f32),
          pltpu.SemaphoreType.DMA,
          pltpu.SemaphoreType.DMA,
          pltpu.SemaphoreType.DMA,
          pltpu.SemaphoreType.DMA,
          pltpu.SemaphoreType.DMA,
          pltpu.SemaphoreType.DMA,
      ],
  )
  def k(t0r, srcr, dstr, zcolr, zslabr, t0pr, t1r, t2r, t3r, atabr,
        acc, deg, s0, s1, d0, d1, m0, m1, obuf, tbuf, abuf,
        is0, is1, gs0, gs1, ss0, ss1):
    sb = (s0, s1)
    db = (d0, d1)
    mb = (m0, m1)
    isem = (is0, is1)
    gsem = (gs0, gs1)
    ssem = (ss0, ss1)

    cid = lax.axis_index("c")
    sid = lax.axis_index("s")
    row0 = sid * ROWS_PER_TILE
    src_off = cid * NN_PAD

    pltpu.sync_copy(zslabr, acc.at[pl.ds(row0, ROWS_PER_TILE)])
    pltpu.sync_copy(zcolr, deg.at[pl.ds(row0, ROWS_PER_TILE)])
    for kk in range(CHUNK // 16):
      obuf[pl.ds(kk * 16, 16)] = jnp.full((16,), 1.0, _f32)
    plsc.subcore_barrier()

    def idx_issue(i, s, need_src=True):
      row = sid * CHUNKS_PER_TILE + 2 * i
      if need_src:
        pltpu.async_copy(srcr.at[pl.ds(row, 2)], sb[s], isem[s])
      pltpu.async_copy(dstr.at[pl.ds(row, 2)], db[s], isem[s])

    def idx_wait(s, need_src=True):
      if need_src:
        pltpu.make_async_copy(srcr.at[pl.ds(0, 2)], sb[s], isem[s]).wait()
      pltpu.make_async_copy(dstr.at[pl.ds(0, 2)], db[s], isem[s]).wait()

    def deg_step(i, s, prefetch=True):
      ns = 1 - s
      idx_wait(s)
      c0 = pltpu.async_copy(obuf, deg.at[sb[s].at[0]], gsem[0], add=True)
      c1 = pltpu.async_copy(obuf, deg.at[sb[s].at[1]], gsem[1], add=True)
      c2 = pltpu.async_copy(obuf, deg.at[db[s].at[0]], ssem[0], add=True)
      c3 = pltpu.async_copy(obuf, deg.at[db[s].at[1]], ssem[1], add=True)
      if prefetch:
        idx_issue(i + 1, ns)
      c0.wait()
      c1.wait()
      c2.wait()
      c3.wait()

    NPAIR = CHUNKS_PER_TILE // 2
    idx_issue(0, 0)

    @pl.loop(0, (NPAIR - 2) // 2)
    def _(kk):
      deg_step(2 * kk, 0)
      deg_step(2 * kk + 1, 1)

    deg_step(NPAIR - 2, 0)
    deg_step(NPAIR - 1, 1, prefetch=False)
    plsc.subcore_barrier()

    @pl.loop(0, ROWS_PER_TILE // SROWS)
    def _(z):
      off = row0 + z * SROWS
      pltpu.sync_copy(deg.at[pl.ds(off, SROWS)], abuf.at[pl.ds(0, SROWS)])
      for kk in range(192 // 16):
        sl = pl.ds(kk * 16, 16)
        abuf[sl] = _rsqrt16(jnp.maximum(abuf[sl], jnp.float32(1.0)))
      pltpu.sync_copy(abuf.at[pl.ds(0, SROWS)], deg.at[pl.ds(off, SROWS)])
      pltpu.sync_copy(abuf.at[pl.ds(0, SROWS)], atabr.at[pl.ds(off, SROWS)])

    def scale_slab(src_hbm, dst_hbm, square):
      @pl.loop(0, ROWS_PER_TILE // SROWS)
      def _(z):
        off = row0 + z * SROWS
        pltpu.sync_copy(src_hbm.at[pl.ds(src_off + off, SROWS)], tbuf)
        pltpu.sync_copy(deg.at[pl.ds(off, SROWS)], abuf.at[pl.ds(0, SROWS)])
        if square:
          for kk in range(192 // 16):
            sl = pl.ds(kk * 16, 16)
            av = abuf[sl]
            abuf[sl] = av * av

        @pl.loop(0, SROWS)
        def _(r):
          asplat = plsc.load_gather(abuf, [jnp.full((16,), r, _i32)])
          for h in range(2):
            sl = pl.ds(h * 16, 16)
            tbuf[r, sl] = tbuf[r, sl] * asplat

        pltpu.sync_copy(tbuf, dst_hbm.at[pl.ds(src_off + off, SROWS)])

    def scale_acc_to(dst_hbm):
      @pl.loop(0, ROWS_PER_TILE // SROWS)
      def _(z):
        off = row0 + z * SROWS
        pltpu.sync_copy(acc.at[pl.ds(off, SROWS)], tbuf)
        pltpu.sync_copy(deg.at[pl.ds(off, SROWS)], abuf.at[pl.ds(0, SROWS)])
        for kk in range(192 // 16):
          sl = pl.ds(kk * 16, 16)
          av = abuf[sl]
          abuf[sl] = av * av

        @pl.loop(0, SROWS)
        def _(r):
          asplat = plsc.load_gather(abuf, [jnp.full((16,), r, _i32)])
          for h in range(2):
            sl = pl.ds(h * 16, 16)
            tbuf[r, sl] = tbuf[r, sl] * asplat

        pltpu.sync_copy(tbuf, dst_hbm.at[pl.ds(src_off + off, SROWS)])

    scale_slab(t0r, t0pr, square=False)
    plsc.subcore_barrier()

    def pair_step(i, s, tprev, prefetch=True):
      ns = 1 - s
      idx_wait(s)
      for r in range(2):
        for kk in range(CHUNK // 16):
          sl = pl.ds(kk * 16, 16)
          sb[s][r, sl] = sb[s][r, sl] + src_off
      ga = pltpu.async_copy(tprev.at[sb[s].at[0]], mb[0], gsem[0])
      gb = pltpu.async_copy(tprev.at[sb[s].at[1]], mb[1], gsem[1])
      if prefetch:
        idx_issue(i + 1, ns)
      ga.wait()
      sa = pltpu.async_copy(mb[0], acc.at[db[s].at[0]], ssem[0], add=True)
      gb.wait()
      sc = pltpu.async_copy(mb[1], acc.at[db[s].at[1]], ssem[1], add=True)
      sa.wait()
      sc.wait()

    def layer(tprev, tout):
      idx_issue(0, 0)

      @pl.loop(0, (NPAIR - 2) // 2)
      def _(kk):
        pair_step(2 * kk, 0, tprev)
        pair_step(2 * kk + 1, 1, tprev)

      pair_step(NPAIR - 2, 0, tprev)
      pair_step(NPAIR - 1, 1, tprev, prefetch=False)
      plsc.subcore_barrier()
      scale_acc_to(tout)
      pltpu.sync_copy(zslabr, acc.at[pl.ds(row0, ROWS_PER_TILE)])
      plsc.subcore_barrier()

    layer(t0pr, t1r)
    layer(t1r, t2r)
    layer(t2r, t3r)

  return k(t0, srcs, dsts, zcol, zslab)


def _readout(user, item, user_emb, item_emb, t0p, t1, t2, t3, atab):
  out_t = (
      jax.ShapeDtypeStruct((BATCH, DH), _f32),
      jax.ShapeDtypeStruct((BATCH, DH), _f32),
      jax.ShapeDtypeStruct((BATCH, DH), _f32),
      jax.ShapeDtypeStruct((BATCH, DH), _f32),
      jax.ShapeDtypeStruct((BATCH,), _f32),
      jax.ShapeDtypeStruct((BATCH,), _f32),
      jax.ShapeDtypeStruct((BATCH, D), _f32),
      jax.ShapeDtypeStruct((BATCH, D), _f32),
  )

  @functools.partial(
      pl.kernel,
      out_type=out_t,
      mesh=_mesh(),
      compiler_params=_sc_params(),
      scratch_types=[
          pltpu.VMEM((PB,), _i32),
          pltpu.VMEM((PB,), _i32),
          pltpu.VMEM((PB,), _i32),
          pltpu.VMEM((PB,), _i32),
          pltpu.VMEM((PB, DH), _f32),
          pltpu.VMEM((PB, DH), _f32),
          pltpu.VMEM((PB, DH), _f32),
          pltpu.VMEM((PB, DH), _f32),
          pltpu.VMEM((PB,), _f32),
          pltpu.VMEM((PB,), _f32),
          pltpu.VMEM((PB, D), _f32),
          pltpu.VMEM((PB, D), _f32),
          pltpu.VMEM((PB, DH), _f32),
      ],
  )
  def k(userr, itemr, uer, ier, t0r, t1r, t2r, t3r, ar,
        uslr, ushr, islr, ishr, aur, air, uegor, iegor,
        ub, ubh, ib, ibh, su_lo, su_hi, si_lo, si_hi, au, ai, eu, ei, tmp):
    cid = lax.axis_index("c")
    sid = lax.axis_index("s")
    wid = cid * TILES + sid
    base = wid * PB

    pltpu.sync_copy(userr.at[pl.ds(base, PB)], ub)
    pltpu.sync_copy(itemr.at[pl.ds(base, PB)], ib)

    pltpu.sync_copy(uer.at[ub], eu)
    pltpu.sync_copy(ier.at[ib], ei)

    for kk in range(PB // 16):
      sl = pl.ds(kk * 16, 16)
      ubh[sl] = ub[sl] + NN_PAD
      ib[sl] = ib[sl] + N_U
      ibh[sl] = ib[sl] + NN_PAD

    pltpu.sync_copy(ar.at[ub], au)
    pltpu.sync_copy(ar.at[ib], ai)

    for idx, accb in ((ub, su_lo), (ubh, su_hi), (ib, si_lo), (ibh, si_hi)):
      pltpu.sync_copy(t0r.at[idx], accb)
      for tr in (t1r, t2r, t3r):
        pltpu.sync_copy(tr.at[idx], tmp)

        @pl.loop(0, PB)
        def _(r):
          for h in range(2):
            sl = pl.ds(h * 16, 16)
            accb[r, sl] = accb[r, sl] + tmp[r, sl]

    pltpu.sync_copy(su_lo, uslr.at[pl.ds(base, PB)])
    pltpu.sync_copy(su_hi, ushr.at[pl.ds(base, PB)])
    pltpu.sync_copy(si_lo, islr.at[pl.ds(base, PB)])
    pltpu.sync_copy(si_hi, ishr.at[pl.ds(base, PB)])
    pltpu.sync_copy(au, aur.at[pl.ds(base, PB)])
    pltpu.sync_copy(ai, air.at[pl.ds(base, PB)])
    pltpu.sync_copy(eu, uegor.at[pl.ds(base, PB)])
    pltpu.sync_copy(ei, iegor.at[pl.ds(base, PB)])

  return k(user, item, user_emb, item_emb, t0p, t1, t2, t3, atab)


def _dot_tc(usl, ush, isl, ish, au, ai):
  def body(a, b, c, d, aur, air, o):
    s = jnp.sum(a[...] * c[...] + b[...] * d[...], axis=1, keepdims=True)
    o[...] = s * jnp.float32(1.0 / 16.0) / (aur[...] * air[...])

  out = pl.pallas_call(
      body,
      out_shape=jax.ShapeDtypeStruct((BATCH, 1), _f32),
  )(usl, ush, isl, ish, au.reshape(BATCH, 1), ai.reshape(BATCH, 1))
  return out.reshape(BATCH)


def kernel(user, u_ir, nbr, item, rate, user_emb, item_emb, edge_index,
           edge_weight):
  del u_ir, nbr, rate, edge_weight
  zrow = jnp.zeros((NN_PAD - NN, DH), _f32)
  t0 = jnp.concatenate(
      [user_emb[:, :DH], item_emb[:, :DH], zrow,
       user_emb[:, DH:], item_emb[:, DH:], zrow],
      axis=0)
  dst = edge_index[0]
  src = edge_index[1]
  e = src.shape[0]
  pad = E_PAD - e
  srcp = jnp.concatenate([src, jnp.full((pad,), NN, _i32)]).reshape(-1, CHUNK)
  dstp = jnp.concatenate([dst, jnp.full((pad,), NN, _i32)]).reshape(-1, CHUNK)
  zcol = jnp.zeros((ROWS_PER_TILE,), _f32)
  zslab = jnp.zeros((ROWS_PER_TILE, DH), _f32)

  t0p, t1, t2, t3, atab = _propagate(t0, srcp, dstp, zcol, zslab)
  usl, ush, isl, ish, au, ai, uego, iego = _readout(
      user, item, user_emb, item_emb, t0p, t1, t2, t3, atab)
  pred = _dot_tc(usl, ush, isl, ish, au, ai)
  return pred, uego, iego


# speedup vs baseline: 9.5207x; 1.1727x over previous
import dataclasses
import functools

import jax
import jax.numpy as jnp
from jax import lax
from jax.experimental import pallas as pl
from jax.experimental.pallas import tpu as pltpu
from jax.experimental.pallas import tpu_sc as plsc

N_U = 15000
N_I = 35000
NN = N_U + N_I
NN_PAD = 50048
D = 64
DH = 32
LAYERS = 3
BATCH = 4096

TILES = 16
CHUNK = 128
CHUNKS_PER_TILE = 392
E_PAD = CHUNKS_PER_TILE * CHUNK * TILES

ROWS_PER_TILE = NN_PAD // TILES
SROWS = 184
PB = BATCH // 32

_f32 = jnp.float32
_i32 = jnp.int32


def _mesh():
  return plsc.VectorSubcoreMesh(core_axis_name="c", subcore_axis_name="s")


def _sc_params():
  cp = pltpu.CompilerParams()
  fields = pltpu.CompilerParams.__dataclass_fields__
  if "needs_layout_passes" in fields:
    cp = dataclasses.replace(cp, needs_layout_passes=False)
  if "use_tc_tiling_on_sc" in fields:
    cp = dataclasses.replace(cp, use_tc_tiling_on_sc=False)
  return cp


def _rsqrt16(x):
  i = plsc.bitcast(x, _i32)
  i = jnp.full((16,), 0x5F3759DF, _i32) - lax.shift_right_logical(i, 1)
  y = plsc.bitcast(i, _f32)
  half = x * jnp.float32(0.5)
  for _ in range(3):
    y = y * (jnp.float32(1.5) - half * y * y)
  return y


def _propagate(t0, srcs, dsts, zcol, zslab):
  out_t = (
      tuple(jax.ShapeDtypeStruct((2 * NN_PAD, DH), _f32) for _ in range(4))
      + (jax.ShapeDtypeStruct((NN_PAD,), _f32),))

  @functools.partial(
      pl.kernel,
      out_type=out_t,
      mesh=_mesh(),
      compiler_params=_sc_params(),
      scratch_types=[
          pltpu.VMEM_SHARED((NN_PAD, DH), _f32),
          pltpu.VMEM_SHARED((NN_PAD,), _f32),
          pltpu.VMEM((2, 2 * CHUNK), _i32),
          pltpu.VMEM((2, 2 * CHUNK), _i32),
          pltpu.VMEM((2, 2 * CHUNK), _i32),
          pltpu.VMEM((2, 2 * CHUNK), _i32),
          pltpu.VMEM((2 * CHUNK, DH), _f32),
          pltpu.VMEM((2 * CHUNK, DH), _f32),
          pltpu.VMEM((2 * CHUNK,), _f32),
          pltpu.VMEM((SROWS, DH), _f32),
          pltpu.VMEM((192,), _f32),
          pltpu.SemaphoreType.DMA,
          pltpu.SemaphoreType.DMA,
          pltpu.SemaphoreType.DMA,
          pltpu.SemaphoreType.DMA,
          pltpu.SemaphoreType.DMA,
          pltpu.SemaphoreType.DMA,
      ],
  )
  def k(t0r, srcr, dstr, zcolr, zslabr, t0pr, t1r, t2r, t3r, atabr,
        acc, deg, s0, s1, d0, d1, m0, m1, obuf, tbuf, abuf,
        is0, is1, gs0, gs1, ss0, ss1):
    sb = (s0, s1)
    db = (d0, d1)
    mb = (m0, m1)
    isem = (is0, is1)
    gsem = (gs0, gs1)
    ssem = (ss0, ss1)

    cid = lax.axis_index("c")
    sid = lax.axis_index("s")
    row0 = sid * ROWS_PER_TILE
    src_off = cid * NN_PAD

    pltpu.sync_copy(zslabr, acc.at[pl.ds(row0, ROWS_PER_TILE)])
    pltpu.sync_copy(zcolr, deg.at[pl.ds(row0, ROWS_PER_TILE)])
    for kk in range(2 * CHUNK // 16):
      obuf[pl.ds(kk * 16, 16)] = jnp.full((16,), 1.0, _f32)
    plsc.subcore_barrier()

    def idx_issue(i, s):
      row = sid * (CHUNKS_PER_TILE // 2) + 2 * i
      pltpu.async_copy(srcr.at[pl.ds(row, 2)], sb[s], isem[s])
      pltpu.async_copy(dstr.at[pl.ds(row, 2)], db[s], isem[s])

    def idx_wait(s):
      pltpu.make_async_copy(srcr.at[pl.ds(0, 2)], sb[s], isem[s]).wait()
      pltpu.make_async_copy(dstr.at[pl.ds(0, 2)], db[s], isem[s]).wait()

    def deg_step(i, s, prefetch=True):
      ns = 1 - s
      idx_wait(s)
      cps = []
      for r in range(2):
        cps.append(pltpu.async_copy(obuf, deg.at[sb[s].at[r]],
                                    gsem[r], add=True))
        cps.append(pltpu.async_copy(obuf, deg.at[db[s].at[r]],
                                    ssem[r], add=True))
      if prefetch:
        idx_issue(i + 1, ns)
      for cp in cps:
        cp.wait()

    NPAIR = CHUNKS_PER_TILE // 4
    idx_issue(0, 0)

    @pl.loop(0, (NPAIR - 2) // 2)
    def _(kk):
      deg_step(2 * kk, 0)
      deg_step(2 * kk + 1, 1)

    deg_step(NPAIR - 2, 0)
    deg_step(NPAIR - 1, 1, prefetch=False)
    plsc.subcore_barrier()

    @pl.loop(0, ROWS_PER_TILE // SROWS)
    def _(z):
      off = row0 + z * SROWS
      pltpu.sync_copy(deg.at[pl.ds(off, SROWS)], abuf.at[pl.ds(0, SROWS)])
      for kk in range(192 // 16):
        sl = pl.ds(kk * 16, 16)
        abuf[sl] = _rsqrt16(jnp.maximum(abuf[sl], jnp.float32(1.0)))
      pltpu.sync_copy(abuf.at[pl.ds(0, SROWS)], deg.at[pl.ds(off, SROWS)])
      pltpu.sync_copy(abuf.at[pl.ds(0, SROWS)], atabr.at[pl.ds(off, SROWS)])

    def scale_slab(src_hbm, dst_hbm, square):
      @pl.loop(0, ROWS_PER_TILE // SROWS)
      def _(z):
        off = row0 + z * SROWS
        pltpu.sync_copy(src_hbm.at[pl.ds(src_off + off, SROWS)], tbuf)
        pltpu.sync_copy(deg.at[pl.ds(off, SROWS)], abuf.at[pl.ds(0, SROWS)])
        if square:
          for kk in range(192 // 16):
            sl = pl.ds(kk * 16, 16)
            av = abuf[sl]
            abuf[sl] = av * av

        @pl.loop(0, SROWS)
        def _(r):
          asplat = plsc.load_gather(abuf, [jnp.full((16,), r, _i32)])
          for h in range(2):
            sl = pl.ds(h * 16, 16)
            tbuf[r, sl] = tbuf[r, sl] * asplat

        pltpu.sync_copy(tbuf, dst_hbm.at[pl.ds(src_off + off, SROWS)])

    def scale_acc_to(dst_hbm):
      @pl.loop(0, ROWS_PER_TILE // SROWS)
      def _(z):
        off = row0 + z * SROWS
        pltpu.sync_copy(acc.at[pl.ds(off, SROWS)], tbuf)
        pltpu.sync_copy(deg.at[pl.ds(off, SROWS)], abuf.at[pl.ds(0, SROWS)])
        for kk in range(192 // 16):
          sl = pl.ds(kk * 16, 16)
          av = abuf[sl]
          abuf[sl] = av * av

        @pl.loop(0, SROWS)
        def _(r):
          asplat = plsc.load_gather(abuf, [jnp.full((16,), r, _i32)])
          for h in range(2):
            sl = pl.ds(h * 16, 16)
            tbuf[r, sl] = tbuf[r, sl] * asplat

        pltpu.sync_copy(tbuf, dst_hbm.at[pl.ds(src_off + off, SROWS)])

    scale_slab(t0r, t0pr, square=False)
    plsc.subcore_barrier()

    def pair_step(i, s, tprev, prefetch=True):
      ns = 1 - s
      idx_wait(s)
      for r in range(2):
        for kk in range(2 * CHUNK // 16):
          sl = pl.ds(kk * 16, 16)
          sb[s][r, sl] = sb[s][r, sl] + src_off
      ga = pltpu.async_copy(tprev.at[sb[s].at[0]], mb[0], gsem[0])
      gb = pltpu.async_copy(tprev.at[sb[s].at[1]], mb[1], gsem[1])
      if prefetch:
        idx_issue(i + 1, ns)
      ga.wait()
      sa = pltpu.async_copy(mb[0], acc.at[db[s].at[0]], ssem[0], add=True)
      gb.wait()
      sc = pltpu.async_copy(mb[1], acc.at[db[s].at[1]], ssem[1], add=True)
      sa.wait()
      sc.wait()

    def layer(tprev, tout):
      idx_issue(0, 0)

      @pl.loop(0, (NPAIR - 2) // 2)
      def _(kk):
        pair_step(2 * kk, 0, tprev)
        pair_step(2 * kk + 1, 1, tprev)

      pair_step(NPAIR - 2, 0, tprev)
      pair_step(NPAIR - 1, 1, tprev, prefetch=False)
      plsc.subcore_barrier()
      scale_acc_to(tout)
      pltpu.sync_copy(zslabr, acc.at[pl.ds(row0, ROWS_PER_TILE)])
      plsc.subcore_barrier()

    layer(t0pr, t1r)
    layer(t1r, t2r)
    layer(t2r, t3r)

  return k(t0, srcs, dsts, zcol, zslab)


def _readout(user, item, user_emb, item_emb, t0p, t1, t2, t3, atab):
  out_t = (
      jax.ShapeDtypeStruct((BATCH, DH), _f32),
      jax.ShapeDtypeStruct((BATCH, DH), _f32),
      jax.ShapeDtypeStruct((BATCH, DH), _f32),
      jax.ShapeDtypeStruct((BATCH, DH), _f32),
      jax.ShapeDtypeStruct((BATCH,), _f32),
      jax.ShapeDtypeStruct((BATCH,), _f32),
      jax.ShapeDtypeStruct((BATCH, D), _f32),
      jax.ShapeDtypeStruct((BATCH, D), _f32),
  )

  @functools.partial(
      pl.kernel,
      out_type=out_t,
      mesh=_mesh(),
      compiler_params=_sc_params(),
      scratch_types=[
          pltpu.VMEM((PB,), _i32),
          pltpu.VMEM((PB,), _i32),
          pltpu.VMEM((PB,), _i32),
          pltpu.VMEM((PB,), _i32),
          pltpu.VMEM((PB, DH), _f32),
          pltpu.VMEM((PB, DH), _f32),
          pltpu.VMEM((PB, DH), _f32),
          pltpu.VMEM((PB, DH), _f32),
          pltpu.VMEM((PB,), _f32),
          pltpu.VMEM((PB,), _f32),
          pltpu.VMEM((PB, D), _f32),
          pltpu.VMEM((PB, D), _f32),
          pltpu.VMEM((PB, DH), _f32),
      ],
  )
  def k(userr, itemr, uer, ier, t0r, t1r, t2r, t3r, ar,
        uslr, ushr, islr, ishr, aur, air, uegor, iegor,
        ub, ubh, ib, ibh, su_lo, su_hi, si_lo, si_hi, au, ai, eu, ei, tmp):
    cid = lax.axis_index("c")
    sid = lax.axis_index("s")
    wid = cid * TILES + sid
    base = wid * PB

    pltpu.sync_copy(userr.at[pl.ds(base, PB)], ub)
    pltpu.sync_copy(itemr.at[pl.ds(base, PB)], ib)

    pltpu.sync_copy(uer.at[ub], eu)
    pltpu.sync_copy(ier.at[ib], ei)

    for kk in range(PB // 16):
      sl = pl.ds(kk * 16, 16)
      ubh[sl] = ub[sl] + NN_PAD
      ib[sl] = ib[sl] + N_U
      ibh[sl] = ib[sl] + NN_PAD

    pltpu.sync_copy(ar.at[ub], au)
    pltpu.sync_copy(ar.at[ib], ai)

    for idx, accb in ((ub, su_lo), (ubh, su_hi), (ib, si_lo), (ibh, si_hi)):
      pltpu.sync_copy(t0r.at[idx], accb)
      for tr in (t1r, t2r, t3r):
        pltpu.sync_copy(tr.at[idx], tmp)

        @pl.loop(0, PB)
        def _(r):
          for h in range(2):
            sl = pl.ds(h * 16, 16)
            accb[r, sl] = accb[r, sl] + tmp[r, sl]

    pltpu.sync_copy(su_lo, uslr.at[pl.ds(base, PB)])
    pltpu.sync_copy(su_hi, ushr.at[pl.ds(base, PB)])
    pltpu.sync_copy(si_lo, islr.at[pl.ds(base, PB)])
    pltpu.sync_copy(si_hi, ishr.at[pl.ds(base, PB)])
    pltpu.sync_copy(au, aur.at[pl.ds(base, PB)])
    pltpu.sync_copy(ai, air.at[pl.ds(base, PB)])
    pltpu.sync_copy(eu, uegor.at[pl.ds(base, PB)])
    pltpu.sync_copy(ei, iegor.at[pl.ds(base, PB)])

  return k(user, item, user_emb, item_emb, t0p, t1, t2, t3, atab)


def _dot_tc(usl, ush, isl, ish, au, ai):
  def body(a, b, c, d, aur, air, o):
    s = jnp.sum(a[...] * c[...] + b[...] * d[...], axis=1, keepdims=True)
    o[...] = s * jnp.float32(1.0 / 16.0) / (aur[...] * air[...])

  out = pl.pallas_call(
      body,
      out_shape=jax.ShapeDtypeStruct((BATCH, 1), _f32),
  )(usl, ush, isl, ish, au.reshape(BATCH, 1), ai.reshape(BATCH, 1))
  return out.reshape(BATCH)


def kernel(user, u_ir, nbr, item, rate, user_emb, item_emb, edge_index,
           edge_weight):
  del u_ir, nbr, rate, edge_weight
  zrow = jnp.zeros((NN_PAD - NN, DH), _f32)
  t0 = jnp.concatenate(
      [user_emb[:, :DH], item_emb[:, :DH], zrow,
       user_emb[:, DH:], item_emb[:, DH:], zrow],
      axis=0)
  dst = edge_index[0]
  src = edge_index[1]
  e = src.shape[0]
  pad = E_PAD - e
  srcp = jnp.concatenate(
      [src, jnp.full((pad,), NN, _i32)]).reshape(-1, 2 * CHUNK)
  dstp = jnp.concatenate(
      [dst, jnp.full((pad,), NN, _i32)]).reshape(-1, 2 * CHUNK)
  zcol = jnp.zeros((ROWS_PER_TILE,), _f32)
  zslab = jnp.zeros((ROWS_PER_TILE, DH), _f32)

  t0p, t1, t2, t3, atab = _propagate(t0, srcp, dstp, zcol, zslab)
  usl, ush, isl, ish, au, ai, uego, iego = _readout(
      user, item, user_emb, item_emb, t0p, t1, t2, t3, atab)
  pred = _dot_tc(usl, ush, isl, ish, au, ai)
  return pred, uego, iego


# speedup vs baseline: 9.5771x; 1.0059x over previous
import dataclasses
import functools

import jax
import jax.numpy as jnp
from jax import lax
from jax.experimental import pallas as pl
from jax.experimental.pallas import tpu as pltpu
from jax.experimental.pallas import tpu_sc as plsc

N_U = 15000
N_I = 35000
NN = N_U + N_I
NN_PAD = 50048
D = 64
DH = 32
LAYERS = 3
BATCH = 4096

TILES = 16
CHUNK = 128
CHUNKS_PER_TILE = 392
E_PAD = CHUNKS_PER_TILE * CHUNK * TILES

ROWS_PER_TILE = NN_PAD // TILES
SROWS = 184
PB = BATCH // 32

_f32 = jnp.float32
_i32 = jnp.int32


def _mesh():
  return plsc.VectorSubcoreMesh(core_axis_name="c", subcore_axis_name="s")


def _sc_params():
  cp = pltpu.CompilerParams()
  fields = pltpu.CompilerParams.__dataclass_fields__
  if "needs_layout_passes" in fields:
    cp = dataclasses.replace(cp, needs_layout_passes=False)
  if "use_tc_tiling_on_sc" in fields:
    cp = dataclasses.replace(cp, use_tc_tiling_on_sc=False)
  return cp


def _rsqrt16(x):
  i = plsc.bitcast(x, _i32)
  i = jnp.full((16,), 0x5F3759DF, _i32) - lax.shift_right_logical(i, 1)
  y = plsc.bitcast(i, _f32)
  half = x * jnp.float32(0.5)
  for _ in range(3):
    y = y * (jnp.float32(1.5) - half * y * y)
  return y


def _propagate(t0, srcs, dsts, zcol, zslab):
  out_t = (
      tuple(jax.ShapeDtypeStruct((2 * NN_PAD, DH), _f32) for _ in range(4))
      + (jax.ShapeDtypeStruct((NN_PAD,), _f32),))

  @functools.partial(
      pl.kernel,
      out_type=out_t,
      mesh=_mesh(),
      compiler_params=_sc_params(),
      scratch_types=[
          pltpu.VMEM_SHARED((NN_PAD, DH), _f32),
          pltpu.VMEM_SHARED((NN_PAD,), _f32),
          pltpu.VMEM((2, 2 * CHUNK), _i32),
          pltpu.VMEM((2, 2 * CHUNK), _i32),
          pltpu.VMEM((2, 2 * CHUNK), _i32),
          pltpu.VMEM((2, 2 * CHUNK), _i32),
          pltpu.VMEM((2 * CHUNK, DH), _f32),
          pltpu.VMEM((2 * CHUNK, DH), _f32),
          pltpu.VMEM((2 * CHUNK,), _f32),
          pltpu.VMEM((SROWS, DH), _f32),
          pltpu.VMEM((384,), _f32),
          pltpu.SemaphoreType.DMA,
          pltpu.SemaphoreType.DMA,
          pltpu.SemaphoreType.DMA,
          pltpu.SemaphoreType.DMA,
          pltpu.SemaphoreType.DMA,
          pltpu.SemaphoreType.DMA,
      ],
  )
  def k(t0r, srcr, dstr, zcolr, zslabr, t0pr, t1r, t2r, t3r, atabr,
        acc, deg, s0, s1, d0, d1, m0, m1, obuf, tbuf, abuf,
        is0, is1, gs0, gs1, ss0, ss1):
    sb = (s0, s1)
    db = (d0, d1)
    mb = (m0, m1)
    isem = (is0, is1)
    gsem = (gs0, gs1)
    ssem = (ss0, ss1)

    cid = lax.axis_index("c")
    sid = lax.axis_index("s")
    row0 = sid * ROWS_PER_TILE
    src_off = cid * NN_PAD

    pltpu.sync_copy(zslabr, acc.at[pl.ds(row0, ROWS_PER_TILE)])
    pltpu.sync_copy(zcolr, deg.at[pl.ds(row0, ROWS_PER_TILE)])
    for kk in range(2 * CHUNK // 16):
      obuf[pl.ds(kk * 16, 16)] = jnp.full((16,), 1.0, _f32)
    plsc.subcore_barrier()

    def idx_issue(i, s):
      row = sid * (CHUNKS_PER_TILE // 2) + 2 * i
      pltpu.async_copy(srcr.at[pl.ds(row, 2)], sb[s], isem[s])
      pltpu.async_copy(dstr.at[pl.ds(row, 2)], db[s], isem[s])

    def idx_wait(s):
      pltpu.make_async_copy(srcr.at[pl.ds(0, 2)], sb[s], isem[s]).wait()
      pltpu.make_async_copy(dstr.at[pl.ds(0, 2)], db[s], isem[s]).wait()

    def deg_step(i, s, prefetch=True):
      ns = 1 - s
      idx_wait(s)
      cps = []
      for r in range(2):
        cps.append(pltpu.async_copy(obuf, deg.at[sb[s].at[r]],
                                    gsem[r], add=True))
        cps.append(pltpu.async_copy(obuf, deg.at[db[s].at[r]],
                                    ssem[r], add=True))
      if prefetch:
        idx_issue(i + 1, ns)
      for cp in cps:
        cp.wait()

    NPAIR = CHUNKS_PER_TILE // 4
    idx_issue(0, 0)

    @pl.loop(0, (NPAIR - 2) // 2)
    def _(kk):
      deg_step(2 * kk, 0)
      deg_step(2 * kk + 1, 1)

    deg_step(NPAIR - 2, 0)
    deg_step(NPAIR - 1, 1, prefetch=False)
    plsc.subcore_barrier()

    NZ = ROWS_PER_TILE // SROWS

    def scale_pass(src_acc, src_hbm, dst_hbm, first):
      @pl.loop(0, ROWS_PER_TILE // SROWS)
      def _(z):
        off = row0 + z * SROWS
        if src_acc:
          c0 = pltpu.async_copy(acc.at[pl.ds(off, SROWS)], tbuf, isem[0])
        else:
          c0 = pltpu.async_copy(src_hbm.at[pl.ds(src_off + off, SROWS)],
                                tbuf, isem[0])
        c1 = pltpu.async_copy(deg.at[pl.ds(off, SROWS)],
                              abuf.at[pl.ds(0, SROWS)], isem[1])
        c1.wait()
        for kk in range(192 // 16):
          sl = pl.ds(kk * 16, 16)
          if first:
            abuf[sl] = _rsqrt16(jnp.maximum(abuf[sl], jnp.float32(1.0)))
          else:
            av = abuf[sl]
            abuf[sl] = av * av
        if first:
          pltpu.sync_copy(abuf.at[pl.ds(0, SROWS)], deg.at[pl.ds(off, SROWS)])
          pltpu.sync_copy(abuf.at[pl.ds(0, SROWS)],
                          atabr.at[pl.ds(off, SROWS)])
        c0.wait()

        @pl.loop(0, SROWS)
        def _(r):
          asplat = plsc.load_gather(abuf, [jnp.full((16,), r, _i32)])
          for h in range(2):
            sl = pl.ds(h * 16, 16)
            tbuf[r, sl] = tbuf[r, sl] * asplat

        pltpu.sync_copy(tbuf, dst_hbm.at[pl.ds(src_off + off, SROWS)])

    scale_pass(False, t0r, t0pr, first=True)
    plsc.subcore_barrier()

    def pair_step(i, s, tprev, prefetch=True):
      ns = 1 - s
      idx_wait(s)
      for r in range(2):
        for kk in range(2 * CHUNK // 16):
          sl = pl.ds(kk * 16, 16)
          sb[s][r, sl] = sb[s][r, sl] + src_off
      ga = pltpu.async_copy(tprev.at[sb[s].at[0]], mb[0], gsem[0])
      gb = pltpu.async_copy(tprev.at[sb[s].at[1]], mb[1], gsem[1])
      if prefetch:
        idx_issue(i + 1, ns)
      ga.wait()
      sa = pltpu.async_copy(mb[0], acc.at[db[s].at[0]], ssem[0], add=True)
      gb.wait()
      sc = pltpu.async_copy(mb[1], acc.at[db[s].at[1]], ssem[1], add=True)
      sa.wait()
      sc.wait()

    def layer(tprev, tout):
      idx_issue(0, 0)

      @pl.loop(0, (NPAIR - 2) // 2)
      def _(kk):
        pair_step(2 * kk, 0, tprev)
        pair_step(2 * kk + 1, 1, tprev)

      pair_step(NPAIR - 2, 0, tprev)
      pair_step(NPAIR - 1, 1, tprev, prefetch=False)
      plsc.subcore_barrier()
      scale_pass(True, None, tout, first=False)
      pltpu.sync_copy(zslabr, acc.at[pl.ds(row0, ROWS_PER_TILE)])
      plsc.subcore_barrier()

    layer(t0pr, t1r)
    layer(t1r, t2r)
    layer(t2r, t3r)

  return k(t0, srcs, dsts, zcol, zslab)


def _readout(user, item, user_emb, item_emb, t0p, t1, t2, t3, atab):
  out_t = (
      jax.ShapeDtypeStruct((BATCH, DH), _f32),
      jax.ShapeDtypeStruct((BATCH, DH), _f32),
      jax.ShapeDtypeStruct((BATCH, DH), _f32),
      jax.ShapeDtypeStruct((BATCH, DH), _f32),
      jax.ShapeDtypeStruct((BATCH,), _f32),
      jax.ShapeDtypeStruct((BATCH,), _f32),
      jax.ShapeDtypeStruct((BATCH, D), _f32),
      jax.ShapeDtypeStruct((BATCH, D), _f32),
  )

  @functools.partial(
      pl.kernel,
      out_type=out_t,
      mesh=_mesh(),
      compiler_params=_sc_params(),
      scratch_types=[
          pltpu.VMEM((PB,), _i32),
          pltpu.VMEM((PB,), _i32),
          pltpu.VMEM((PB,), _i32),
          pltpu.VMEM((PB,), _i32),
          pltpu.VMEM((PB, DH), _f32),
          pltpu.VMEM((PB, DH), _f32),
          pltpu.VMEM((PB, DH), _f32),
          pltpu.VMEM((PB, DH), _f32),
          pltpu.VMEM((PB,), _f32),
          pltpu.VMEM((PB,), _f32),
          pltpu.VMEM((PB, D), _f32),
          pltpu.VMEM((PB, D), _f32),
          pltpu.VMEM((PB, DH), _f32),
      ],
  )
  def k(userr, itemr, uer, ier, t0r, t1r, t2r, t3r, ar,
        uslr, ushr, islr, ishr, aur, air, uegor, iegor,
        ub, ubh, ib, ibh, su_lo, su_hi, si_lo, si_hi, au, ai, eu, ei, tmp):
    cid = lax.axis_index("c")
    sid = lax.axis_index("s")
    wid = cid * TILES + sid
    base = wid * PB

    pltpu.sync_copy(userr.at[pl.ds(base, PB)], ub)
    pltpu.sync_copy(itemr.at[pl.ds(base, PB)], ib)

    pltpu.sync_copy(uer.at[ub], eu)
    pltpu.sync_copy(ier.at[ib], ei)

    for kk in range(PB // 16):
      sl = pl.ds(kk * 16, 16)
      ubh[sl] = ub[sl] + NN_PAD
      ib[sl] = ib[sl] + N_U
      ibh[sl] = ib[sl] + NN_PAD

    pltpu.sync_copy(ar.at[ub], au)
    pltpu.sync_copy(ar.at[ib], ai)

    for idx, accb in ((ub, su_lo), (ubh, su_hi), (ib, si_lo), (ibh, si_hi)):
      pltpu.sync_copy(t0r.at[idx], accb)
      for tr in (t1r, t2r, t3r):
        pltpu.sync_copy(tr.at[idx], tmp)

        @pl.loop(0, PB)
        def _(r):
          for h in range(2):
            sl = pl.ds(h * 16, 16)
            accb[r, sl] = accb[r, sl] + tmp[r, sl]

    pltpu.sync_copy(su_lo, uslr.at[pl.ds(base, PB)])
    pltpu.sync_copy(su_hi, ushr.at[pl.ds(base, PB)])
    pltpu.sync_copy(si_lo, islr.at[pl.ds(base, PB)])
    pltpu.sync_copy(si_hi, ishr.at[pl.ds(base, PB)])
    pltpu.sync_copy(au, aur.at[pl.ds(base, PB)])
    pltpu.sync_copy(ai, air.at[pl.ds(base, PB)])
    pltpu.sync_copy(eu, uegor.at[pl.ds(base, PB)])
    pltpu.sync_copy(ei, iegor.at[pl.ds(base, PB)])

  return k(user, item, user_emb, item_emb, t0p, t1, t2, t3, atab)


def _dot_tc(usl, ush, isl, ish, au, ai):
  def body(a, b, c, d, aur, air, o):
    s = jnp.sum(a[...] * c[...] + b[...] * d[...], axis=1, keepdims=True)
    o[...] = s * jnp.float32(1.0 / 16.0) / (aur[...] * air[...])

  out = pl.pallas_call(
      body,
      out_shape=jax.ShapeDtypeStruct((BATCH, 1), _f32),
  )(usl, ush, isl, ish, au.reshape(BATCH, 1), ai.reshape(BATCH, 1))
  return out.reshape(BATCH)


def kernel(user, u_ir, nbr, item, rate, user_emb, item_emb, edge_index,
           edge_weight):
  del u_ir, nbr, rate, edge_weight
  zrow = jnp.zeros((NN_PAD - NN, DH), _f32)
  t0 = jnp.concatenate(
      [user_emb[:, :DH], item_emb[:, :DH], zrow,
       user_emb[:, DH:], item_emb[:, DH:], zrow],
      axis=0)
  dst = edge_index[0]
  src = edge_index[1]
  e = src.shape[0]
  pad = E_PAD - e
  srcp = jnp.concatenate(
      [src, jnp.full((pad,), NN, _i32)]).reshape(-1, 2 * CHUNK)
  dstp = jnp.concatenate(
      [dst, jnp.full((pad,), NN, _i32)]).reshape(-1, 2 * CHUNK)
  zcol = jnp.zeros((ROWS_PER_TILE,), _f32)
  zslab = jnp.zeros((ROWS_PER_TILE, DH), _f32)

  t0p, t1, t2, t3, atab = _propagate(t0, srcp, dstp, zcol, zslab)
  usl, ush, isl, ish, au, ai, uego, iego = _readout(
      user, item, user_emb, item_emb, t0p, t1, t2, t3, atab)
  pred = _dot_tc(usl, ush, isl, ish, au, ai)
  return pred, uego, iego
